# baseline (speedup 1.0000x reference)
import functools

import jax
import jax.numpy as jnp
from jax import lax
from jax.experimental import pallas as pl
from jax.experimental.pallas import tpu as pltpu
from jax.experimental.pallas import tpu_sc as plsc

VOCAB_ROWS = 1000000
DIM = 64
SCALE = 8.0

NC = 2
NS = 16
NW = NC * NS

CHUNK = 128
LANES = 16


def _body(x_hbm, table_hbm, out_hbm, idx_v, rows_v, sem_in):
    wid = lax.axis_index("s") * NC + lax.axis_index("c")
    n_chunks = x_hbm.shape[1]
    row_base = wid * (n_chunks * CHUNK)

    pltpu.sync_copy(x_hbm.at[wid], idx_v)

    def step(j, _):
        pltpu.make_async_copy(
            table_hbm.at[idx_v.at[j]], rows_v, sem_in).start()
        pltpu.make_async_copy(
            table_hbm.at[idx_v.at[j]], rows_v, sem_in).wait()

        def scale_row(i, _):
            for c in range(DIM // LANES):
                sl = pl.ds(c * LANES, LANES)
                rows_v[i, sl] = rows_v[i, sl] * SCALE
            return 0
        lax.fori_loop(0, CHUNK, scale_row, 0, unroll=4)

        pltpu.sync_copy(rows_v,
                        out_hbm.at[pl.ds(row_base + j * CHUNK, CHUNK)])
        return 0

    lax.fori_loop(0, n_chunks, step, 0)


def kernel(x, table):
    b0, b1 = x.shape
    total = b0 * b1
    n_chunks = total // (NW * CHUNK)
    xf = x.reshape(NW, n_chunks, CHUNK).astype(jnp.int32)

    mesh = plsc.VectorSubcoreMesh(core_axis_name="c", subcore_axis_name="s")
    run = pl.kernel(
        _body,
        out_type=jax.ShapeDtypeStruct((total, DIM), jnp.float32),
        mesh=mesh,
        scratch_types=[
            pltpu.VMEM((n_chunks, CHUNK), jnp.int32),
            pltpu.VMEM((CHUNK, DIM), jnp.float32),
            pltpu.SemaphoreType.DMA,
        ],
        compiler_params=pltpu.CompilerParams(use_tc_tiling_on_sc=False),
    )
    out = run(xf, table)
    return out.reshape(b0, b1, DIM)
